# baseline (speedup 1.0000x reference)
import functools

import jax
import jax.numpy as jnp
from jax import lax
from jax.experimental import pallas as pl
from jax.experimental.pallas import tpu as pltpu
from jax.experimental.pallas import tpu_sc as plsc

D = 768
MAXPOS = 1025
B = 64
L = 1024
N = B * L
EPS = 1e-12

NC, NS, LANES = 2, 16, 16
NW = NC * NS
TPW = N // NW
K = 64
NCHUNK = TPW // K
NV = D // LANES
INV_D = 1.0 / D


def _comb_body(pos_ref, type_ref, out_ref):
    row = lax.broadcasted_iota(jnp.int32, (MAXPOS, D), 0)
    t0 = type_ref[0:1, :]
    t1 = type_ref[1:2, :]
    out_ref[...] = pos_ref[...] + jnp.where(row == 0, t0, t1)


_comb_call = pl.pallas_call(
    _comb_body,
    out_shape=jax.ShapeDtypeStruct((MAXPOS, D), jnp.float32),
)


def _rsqrt16(a):
    yi = plsc.bitcast(a, jnp.int32)
    magic = jnp.full((LANES,), 0x5F3759DF, dtype=jnp.int32)
    y = plsc.bitcast(magic - lax.shift_right_logical(yi, 1), jnp.float32)
    half = a * 0.5
    for _ in range(3):
        y = y * (1.5 - half * y * y)
    return y


def _splat_i32(x):
    return jnp.full((LANES,), x, dtype=jnp.int32)


_mesh = plsc.VectorSubcoreMesh(core_axis_name="c", subcore_axis_name="s")


@functools.partial(
    pl.kernel,
    mesh=_mesh,
    compiler_params=pltpu.CompilerParams(needs_layout_passes=False),
    out_type=jax.ShapeDtypeStruct((N, D), jnp.float32),
    scratch_types=[
        pltpu.VMEM((K,), jnp.int32),
        pltpu.VMEM((K,), jnp.int32),
        pltpu.VMEM((K, D), jnp.float32),
        pltpu.VMEM((K, D), jnp.float32),
        pltpu.VMEM((K, LANES), jnp.float32),
        pltpu.VMEM((K, LANES), jnp.float32),
        pltpu.VMEM((K,), jnp.float32),
        pltpu.VMEM((K,), jnp.float32),
        pltpu.VMEM((D,), jnp.float32),
        pltpu.VMEM((D,), jnp.float32),
        pltpu.SemaphoreType.DMA,
        pltpu.SemaphoreType.DMA,
    ],
)
def _sc_embed(ids_hbm, tok_hbm, comb_hbm, gamma_hbm, beta_hbm, out_hbm,
              ids_v, pidx_v, tok_v, comb_v, sb_v, ssb_v, mean_v, inv_v,
              gam_v, bet_v, sem_a, sem_b):
    wid = lax.axis_index("s") * NC + lax.axis_index("c")
    base = wid * TPW
    pltpu.sync_copy(gamma_hbm, gam_v)
    pltpu.sync_copy(beta_hbm, bet_v)

    def chunk_body(c, carry):
        cbase = base + c * K
        posbase = lax.rem(cbase, L)
        pltpu.sync_copy(ids_hbm.at[pl.ds(cbase, K)], ids_v)

        def idx_body(g, carry2):
            idv = ids_v[pl.ds(g * LANES, LANES)]
            pos = posbase + 1 + g * LANES + lax.iota(jnp.int32, LANES)
            pidx_v[pl.ds(g * LANES, LANES)] = jnp.where(idv == 0, 0, pos)
            return carry2

        lax.fori_loop(0, K // LANES, idx_body, 0)

        cp_tok = pltpu.async_copy(tok_hbm.at[ids_v], tok_v, sem_a)
        cp_comb = pltpu.async_copy(comb_hbm.at[pidx_v], comb_v, sem_b)
        cp_tok.wait()
        cp_comb.wait()

        zero = jnp.zeros((LANES,), jnp.float32)

        def pass1(t, carry3):
            def dim_body(v, acc):
                s, ss = acc
                x = tok_v[t, pl.ds(v * LANES, LANES)] + comb_v[t, pl.ds(v * LANES, LANES)]
                tok_v[t, pl.ds(v * LANES, LANES)] = x
                return (s + x, ss + x * x)

            s, ss = lax.fori_loop(0, NV, dim_body, (zero, zero))
            sb_v[t, :] = s
            ssb_v[t, :] = ss
            return carry3

        lax.fori_loop(0, K, pass1, 0)

        def reduce_body(g, carry3):
            rows = g * LANES + lax.iota(jnp.int32, LANES)

            def jloop(j, acc):
                st, sst = acc
                col = _splat_i32(j)
                st = st + plsc.load_gather(sb_v, [rows, col])
                sst = sst + plsc.load_gather(ssb_v, [rows, col])
                return (st, sst)

            s_tot, ss_tot = lax.fori_loop(0, LANES, jloop, (zero, zero))
            mean = s_tot * INV_D
            var = ss_tot * INV_D - mean * mean
            mean_v[pl.ds(g * LANES, LANES)] = mean
            inv_v[pl.ds(g * LANES, LANES)] = _rsqrt16(var + EPS)
            return carry3

        lax.fori_loop(0, K // LANES, reduce_body, 0)

        def pass2(t, carry3):
            mv = plsc.load_gather(mean_v, [_splat_i32(t)])
            iv = plsc.load_gather(inv_v, [_splat_i32(t)])

            def dim_body(v, carry4):
                x = tok_v[t, pl.ds(v * LANES, LANES)]
                g = gam_v[pl.ds(v * LANES, LANES)]
                b = bet_v[pl.ds(v * LANES, LANES)]
                comb_v[t, pl.ds(v * LANES, LANES)] = (x - mv) * iv * g + b
                return carry4

            lax.fori_loop(0, NV, dim_body, 0)
            return carry3

        lax.fori_loop(0, K, pass2, 0)
        pltpu.sync_copy(comb_v, out_hbm.at[pl.ds(cbase, K)])
        return carry

    lax.fori_loop(0, NCHUNK, chunk_body, 0)


def kernel(input_ids, tok_emb, pos_emb, type_emb, gamma, beta):
    comb = _comb_call(pos_emb, type_emb)
    ids = input_ids.reshape(-1).astype(jnp.int32)
    out = _sc_embed(ids, tok_emb, comb, gamma, beta)
    return out.reshape(input_ids.shape[0], input_ids.shape[1], D)


# speedup vs baseline: 1.3823x; 1.3823x over previous
import functools

import jax
import jax.numpy as jnp
from jax import lax
from jax.experimental import pallas as pl
from jax.experimental.pallas import tpu as pltpu
from jax.experimental.pallas import tpu_sc as plsc

D = 768
MAXPOS = 1025
B = 64
L = 1024
N = B * L
EPS = 1e-12

NC, NS, LANES = 2, 16, 16
NW = NC * NS
TPW = N // NW
K = 64
NCHUNK = TPW // K
NV = D // LANES
INV_D = 1.0 / D


def _comb_body(pos_ref, type_ref, out_ref):
    row = lax.broadcasted_iota(jnp.int32, (MAXPOS, D), 0)
    t0 = type_ref[0:1, :]
    t1 = type_ref[1:2, :]
    out_ref[...] = pos_ref[...] + jnp.where(row == 0, t0, t1)


_comb_call = pl.pallas_call(
    _comb_body,
    out_shape=jax.ShapeDtypeStruct((MAXPOS, D), jnp.float32),
)


def _rsqrt16(a):
    yi = plsc.bitcast(a, jnp.int32)
    magic = jnp.full((LANES,), 0x5F3759DF, dtype=jnp.int32)
    y = plsc.bitcast(magic - lax.shift_right_logical(yi, 1), jnp.float32)
    half = a * 0.5
    for _ in range(3):
        y = y * (1.5 - half * y * y)
    return y


def _splat_i32(x):
    return jnp.full((LANES,), x, dtype=jnp.int32)


_mesh = plsc.VectorSubcoreMesh(core_axis_name="c", subcore_axis_name="s")


@functools.partial(
    pl.kernel,
    mesh=_mesh,
    compiler_params=pltpu.CompilerParams(needs_layout_passes=False),
    out_type=jax.ShapeDtypeStruct((N, D), jnp.float32),
    scratch_types=[
        pltpu.VMEM((K,), jnp.int32),
        pltpu.VMEM((K,), jnp.int32),
        pltpu.VMEM((K, D), jnp.float32),
        pltpu.VMEM((K, D), jnp.float32),
        pltpu.VMEM((K, LANES), jnp.float32),
        pltpu.VMEM((K, LANES), jnp.float32),
        pltpu.VMEM((K,), jnp.float32),
        pltpu.VMEM((K,), jnp.float32),
        pltpu.VMEM((D,), jnp.float32),
        pltpu.VMEM((D,), jnp.float32),
        pltpu.SemaphoreType.DMA,
        pltpu.SemaphoreType.DMA,
    ],
)
def _sc_embed(ids_hbm, tok_hbm, comb_hbm, gamma_hbm, beta_hbm, out_hbm,
              ids_v, pidx_v, tok_v, comb_v, sb_v, ssb_v, mean_v, inv_v,
              gam_v, bet_v, sem_a, sem_b):
    wid = lax.axis_index("s") * NC + lax.axis_index("c")
    base = wid * TPW
    pltpu.sync_copy(gamma_hbm, gam_v)
    pltpu.sync_copy(beta_hbm, bet_v)

    def chunk_body(c, carry):
        cbase = base + c * K
        posbase = lax.rem(cbase, L)
        pltpu.sync_copy(ids_hbm.at[pl.ds(cbase, K)], ids_v)

        for g in range(K // LANES):
            idv = ids_v[pl.ds(g * LANES, LANES)]
            pos = posbase + 1 + g * LANES + lax.iota(jnp.int32, LANES)
            pidx_v[pl.ds(g * LANES, LANES)] = jnp.where(idv == 0, 0, pos)

        cp_tok = pltpu.async_copy(tok_hbm.at[ids_v], tok_v, sem_a)
        cp_comb = pltpu.async_copy(comb_hbm.at[pidx_v], comb_v, sem_b)
        cp_tok.wait()
        cp_comb.wait()

        zero = jnp.zeros((LANES,), jnp.float32)

        def pass1(t, carry3):
            s = zero
            ss = zero
            for v in range(NV):
                x = tok_v[t, pl.ds(v * LANES, LANES)] + comb_v[t, pl.ds(v * LANES, LANES)]
                tok_v[t, pl.ds(v * LANES, LANES)] = x
                s = s + x
                ss = ss + x * x
            sb_v[t, :] = s
            ssb_v[t, :] = ss
            return carry3

        lax.fori_loop(0, K, pass1, 0)

        for g in range(K // LANES):
            rows = g * LANES + lax.iota(jnp.int32, LANES)
            s_tot = zero
            ss_tot = zero
            for j in range(LANES):
                col = _splat_i32(j)
                s_tot = s_tot + plsc.load_gather(sb_v, [rows, col])
                ss_tot = ss_tot + plsc.load_gather(ssb_v, [rows, col])
            mean = s_tot * INV_D
            var = ss_tot * INV_D - mean * mean
            mean_v[pl.ds(g * LANES, LANES)] = mean
            inv_v[pl.ds(g * LANES, LANES)] = _rsqrt16(var + EPS)

        def pass2(t, carry3):
            mv = plsc.load_gather(mean_v, [_splat_i32(t)])
            iv = plsc.load_gather(inv_v, [_splat_i32(t)])
            for v in range(NV):
                x = tok_v[t, pl.ds(v * LANES, LANES)]
                g = gam_v[pl.ds(v * LANES, LANES)]
                b = bet_v[pl.ds(v * LANES, LANES)]
                comb_v[t, pl.ds(v * LANES, LANES)] = (x - mv) * iv * g + b
            return carry3

        lax.fori_loop(0, K, pass2, 0)
        pltpu.sync_copy(comb_v, out_hbm.at[pl.ds(cbase, K)])
        return carry

    lax.fori_loop(0, NCHUNK, chunk_body, 0)


def kernel(input_ids, tok_emb, pos_emb, type_emb, gamma, beta):
    comb = _comb_call(pos_emb, type_emb)
    ids = input_ids.reshape(-1).astype(jnp.int32)
    out = _sc_embed(ids, tok_emb, comb, gamma, beta)
    return out.reshape(input_ids.shape[0], input_ids.shape[1], D)


# speedup vs baseline: 1.6456x; 1.1905x over previous
import functools

import jax
import jax.numpy as jnp
from jax import lax
from jax.experimental import pallas as pl
from jax.experimental.pallas import tpu as pltpu
from jax.experimental.pallas import tpu_sc as plsc

D = 768
MAXPOS = 1025
B = 64
L = 1024
N = B * L
EPS = 1e-12

NC, NS, LANES = 2, 16, 16
NW = NC * NS
TPW = N // NW
K = 32
NCHUNK = TPW // K
NV = D // LANES
INV_D = 1.0 / D


def _comb_body(pos_ref, type_ref, out_ref):
    row = lax.broadcasted_iota(jnp.int32, (MAXPOS, D), 0)
    t0 = type_ref[0:1, :]
    t1 = type_ref[1:2, :]
    out_ref[...] = pos_ref[...] + jnp.where(row == 0, t0, t1)


_comb_call = pl.pallas_call(
    _comb_body,
    out_shape=jax.ShapeDtypeStruct((MAXPOS, D), jnp.float32),
)


def _rsqrt16(a):
    yi = plsc.bitcast(a, jnp.int32)
    magic = jnp.full((LANES,), 0x5F3759DF, dtype=jnp.int32)
    y = plsc.bitcast(magic - lax.shift_right_logical(yi, 1), jnp.float32)
    half = a * 0.5
    for _ in range(3):
        y = y * (1.5 - half * y * y)
    return y


def _splat_i32(x):
    return jnp.full((LANES,), x, dtype=jnp.int32)


_mesh = plsc.VectorSubcoreMesh(core_axis_name="c", subcore_axis_name="s")


@functools.partial(
    pl.kernel,
    mesh=_mesh,
    compiler_params=pltpu.CompilerParams(needs_layout_passes=False),
    out_type=jax.ShapeDtypeStruct((N, D), jnp.float32),
    scratch_types=[
        pltpu.VMEM((2, K), jnp.int32),
        pltpu.VMEM((2, K), jnp.int32),
        pltpu.VMEM((2, K, D), jnp.float32),
        pltpu.VMEM((2, K, D), jnp.float32),
        pltpu.VMEM((K, LANES), jnp.float32),
        pltpu.VMEM((K, LANES), jnp.float32),
        pltpu.VMEM((K,), jnp.float32),
        pltpu.VMEM((K,), jnp.float32),
        pltpu.VMEM((D,), jnp.float32),
        pltpu.VMEM((D,), jnp.float32),
        pltpu.SemaphoreType.DMA,
        pltpu.SemaphoreType.DMA,
        pltpu.SemaphoreType.DMA,
        pltpu.SemaphoreType.DMA,
        pltpu.SemaphoreType.DMA,
        pltpu.SemaphoreType.DMA,
    ],
)
def _sc_embed(ids_hbm, tok_hbm, comb_hbm, gamma_hbm, beta_hbm, out_hbm,
              ids_v, pidx_v, tok_v, comb_v, sb_v, ssb_v, mean_v, inv_v,
              gam_v, bet_v, st0, st1, sc0, sc1, so0, so1):
    wid = lax.axis_index("s") * NC + lax.axis_index("c")
    base = wid * TPW
    sem_tok = (st0, st1)
    sem_comb = (sc0, sc1)
    sem_out = (so0, so1)
    pltpu.sync_copy(gamma_hbm, gam_v)
    pltpu.sync_copy(beta_hbm, bet_v)

    zero = jnp.zeros((LANES,), jnp.float32)

    def fetch_ids(c, p):
        cbase = base + c * K
        posbase = lax.rem(cbase, L)
        iv = ids_v.at[p]
        pv = pidx_v.at[p]
        pltpu.sync_copy(ids_hbm.at[pl.ds(cbase, K)], iv)
        for g in range(K // LANES):
            idv = iv[pl.ds(g * LANES, LANES)]
            pos = posbase + 1 + g * LANES + lax.iota(jnp.int32, LANES)
            pv[pl.ds(g * LANES, LANES)] = jnp.where(idv == 0, 0, pos)

    def fire_gathers(p):
        pltpu.async_copy(tok_hbm.at[ids_v.at[p]], tok_v.at[p], sem_tok[p])
        pltpu.async_copy(comb_hbm.at[pidx_v.at[p]], comb_v.at[p], sem_comb[p])

    def wait_gathers(p):
        pltpu.make_async_copy(tok_hbm.at[pl.ds(0, K)], tok_v.at[p], sem_tok[p]).wait()
        pltpu.make_async_copy(comb_hbm.at[pl.ds(0, K)], comb_v.at[p], sem_comb[p]).wait()

    def wait_out(p):
        pltpu.make_async_copy(tok_hbm.at[pl.ds(0, K)], tok_v.at[p], sem_out[p]).wait()

    def compute_chunk(p):
        tv = tok_v.at[p]
        cv = comb_v.at[p]

        def pass1(t, carry):
            s = zero
            ss = zero
            for v in range(NV):
                x = tv[t, pl.ds(v * LANES, LANES)] + cv[t, pl.ds(v * LANES, LANES)]
                tv[t, pl.ds(v * LANES, LANES)] = x
                s = s + x
                ss = ss + x * x
            sb_v[t, :] = s
            ssb_v[t, :] = ss
            return carry

        lax.fori_loop(0, K, pass1, 0)

        for g in range(K // LANES):
            rows = g * LANES + lax.iota(jnp.int32, LANES)
            s_tot = zero
            ss_tot = zero
            for j in range(LANES):
                col = _splat_i32(j)
                s_tot = s_tot + plsc.load_gather(sb_v, [rows, col])
                ss_tot = ss_tot + plsc.load_gather(ssb_v, [rows, col])
            mean = s_tot * INV_D
            var = ss_tot * INV_D - mean * mean
            mean_v[pl.ds(g * LANES, LANES)] = mean
            inv_v[pl.ds(g * LANES, LANES)] = _rsqrt16(var + EPS)

        def pass2(t, carry):
            mv = plsc.load_gather(mean_v, [_splat_i32(t)])
            iv = plsc.load_gather(inv_v, [_splat_i32(t)])
            for v in range(NV):
                x = tv[t, pl.ds(v * LANES, LANES)]
                g = gam_v[pl.ds(v * LANES, LANES)]
                b = bet_v[pl.ds(v * LANES, LANES)]
                tv[t, pl.ds(v * LANES, LANES)] = (x - mv) * iv * g + b
            return carry

        lax.fori_loop(0, K, pass2, 0)

    fetch_ids(0, 0)
    fire_gathers(0)

    def outer(cc, carry):
        for p in (0, 1):
            c = cc * 2 + p
            @pl.when(c + 1 < NCHUNK)
            def _prefetch():
                fetch_ids(c + 1, 1 - p)

                @pl.when(c >= 1)
                def _drain_out():
                    wait_out(1 - p)

                fire_gathers(1 - p)

            wait_gathers(p)
            compute_chunk(p)
            cbase = base + c * K
            pltpu.async_copy(tok_v.at[p], out_hbm.at[pl.ds(cbase, K)], sem_out[p])
        return carry

    lax.fori_loop(0, NCHUNK // 2, outer, 0)
    wait_out(0)
    wait_out(1)


def kernel(input_ids, tok_emb, pos_emb, type_emb, gamma, beta):
    comb = _comb_call(pos_emb, type_emb)
    ids = input_ids.reshape(-1).astype(jnp.int32)
    out = _sc_embed(ids, tok_emb, comb, gamma, beta)
    return out.reshape(input_ids.shape[0], input_ids.shape[1], D)


# speedup vs baseline: 3.4186x; 2.0774x over previous
import functools

import jax
import jax.numpy as jnp
from jax import lax
from jax.experimental import pallas as pl
from jax.experimental.pallas import tpu as pltpu
from jax.experimental.pallas import tpu_sc as plsc

D = 768
MAXPOS = 1025
B = 64
L = 1024
N = B * L
EPS = 1e-12

NC, NS, LANES = 2, 16, 16
NW = NC * NS
TPW = N // NW
K = 32
NCHUNK = TPW // K
NV = D // LANES
INV_D = 1.0 / D


def _comb_body(pos_ref, type_ref, out_ref):
    row = lax.broadcasted_iota(jnp.int32, (MAXPOS, D), 0)
    t0 = type_ref[0:1, :]
    t1 = type_ref[1:2, :]
    out_ref[...] = pos_ref[...] + jnp.where(row == 0, t0, t1)


_comb_call = pl.pallas_call(
    _comb_body,
    out_shape=jax.ShapeDtypeStruct((MAXPOS, D), jnp.float32),
)


def _rsqrt16(a):
    yi = plsc.bitcast(a, jnp.int32)
    magic = jnp.full((LANES,), 0x5F3759DF, dtype=jnp.int32)
    y = plsc.bitcast(magic - lax.shift_right_logical(yi, 1), jnp.float32)
    half = a * 0.5
    for _ in range(3):
        y = y * (1.5 - half * y * y)
    return y


def _splat_i32(x):
    return jnp.full((LANES,), x, dtype=jnp.int32)


_mesh = plsc.VectorSubcoreMesh(core_axis_name="c", subcore_axis_name="s")


@functools.partial(
    pl.kernel,
    mesh=_mesh,
    compiler_params=pltpu.CompilerParams(needs_layout_passes=False),
    out_type=jax.ShapeDtypeStruct((N, D), jnp.float32),
    scratch_types=[
        pltpu.VMEM((2, K), jnp.int32),
        pltpu.VMEM((2, K), jnp.int32),
        pltpu.VMEM((2, K, D), jnp.float32),
        pltpu.VMEM((2, K, D), jnp.float32),
        pltpu.VMEM((K, LANES), jnp.float32),
        pltpu.VMEM((K, LANES), jnp.float32),
        pltpu.VMEM((K,), jnp.float32),
        pltpu.VMEM((K,), jnp.float32),
        pltpu.VMEM((D,), jnp.float32),
        pltpu.VMEM((D,), jnp.float32),
        pltpu.SemaphoreType.DMA,
        pltpu.SemaphoreType.DMA,
        pltpu.SemaphoreType.DMA,
        pltpu.SemaphoreType.DMA,
        pltpu.SemaphoreType.DMA,
        pltpu.SemaphoreType.DMA,
    ],
)
def _sc_embed(ids_hbm, tok_hbm, comb_hbm, gamma_hbm, beta_hbm, out_hbm,
              ids_v, pidx_v, tok_v, comb_v, sb_v, ssb_v, mean_v, inv_v,
              gam_v, bet_v, st0, st1, sc0, sc1, so0, so1):
    wid = lax.axis_index("s") * NC + lax.axis_index("c")
    base = wid * TPW
    sem_tok = (st0, st1)
    sem_comb = (sc0, sc1)
    sem_out = (so0, so1)
    pltpu.sync_copy(gamma_hbm, gam_v)
    pltpu.sync_copy(beta_hbm, bet_v)

    zero = jnp.zeros((LANES,), jnp.float32)

    def fetch_ids(c, p):
        cbase = base + c * K
        posbase = lax.rem(cbase, L)
        iv = ids_v.at[p]
        pv = pidx_v.at[p]
        pltpu.sync_copy(ids_hbm.at[pl.ds(cbase, K)], iv)
        for g in range(K // LANES):
            idv = iv[pl.ds(g * LANES, LANES)]
            pos = posbase + 1 + g * LANES + lax.iota(jnp.int32, LANES)
            pv[pl.ds(g * LANES, LANES)] = jnp.where(idv == 0, 0, pos)

    def fire_gathers(p):
        pltpu.async_copy(tok_hbm.at[ids_v.at[p]], tok_v.at[p], sem_tok[p])
        pltpu.async_copy(comb_hbm.at[pidx_v.at[p]], comb_v.at[p], sem_comb[p])

    def wait_gathers(p):
        pltpu.make_async_copy(tok_hbm.at[pl.ds(0, K)], tok_v.at[p], sem_tok[p]).wait()
        pltpu.make_async_copy(comb_hbm.at[pl.ds(0, K)], comb_v.at[p], sem_comb[p]).wait()

    def wait_out(p):
        pltpu.make_async_copy(tok_hbm.at[pl.ds(0, K)], tok_v.at[p], sem_out[p]).wait()

    def compute_chunk(p):
        tv = tok_v.at[p]
        cv = comb_v.at[p]

        def pass1(t, carry):
            s = zero
            ss = zero
            for v in range(NV):
                x = tv[t, pl.ds(v * LANES, LANES)] + cv[t, pl.ds(v * LANES, LANES)]
                tv[t, pl.ds(v * LANES, LANES)] = x
                s = s + x
                ss = ss + x * x
            sb_v[t, :] = s
            ssb_v[t, :] = ss
            return carry

        lax.fori_loop(0, K, pass1, 0)

        for g in range(K // LANES):
            rows = g * LANES + lax.iota(jnp.int32, LANES)
            s_tot = zero
            ss_tot = zero
            for j in range(LANES):
                col = _splat_i32(j)
                s_tot = s_tot + plsc.load_gather(sb_v, [rows, col])
                ss_tot = ss_tot + plsc.load_gather(ssb_v, [rows, col])
            mean = s_tot * INV_D
            var = ss_tot * INV_D - mean * mean
            mean_v[pl.ds(g * LANES, LANES)] = mean
            inv_v[pl.ds(g * LANES, LANES)] = _rsqrt16(var + EPS)

        NBLK = 4
        VB = NV // NBLK
        for blk in range(NBLK):
            gs = [gam_v[pl.ds((blk * VB + v) * LANES, LANES)] for v in range(VB)]
            bs = [bet_v[pl.ds((blk * VB + v) * LANES, LANES)] for v in range(VB)]

            def pass2(t, carry, _gs=gs, _bs=bs, _blk=blk):
                mv = plsc.load_gather(mean_v, [_splat_i32(t)])
                iv = plsc.load_gather(inv_v, [_splat_i32(t)])
                for v in range(VB):
                    off = (_blk * VB + v) * LANES
                    x = tv[t, pl.ds(off, LANES)]
                    tv[t, pl.ds(off, LANES)] = (x - mv) * iv * _gs[v] + _bs[v]
                return carry

            lax.fori_loop(0, K, pass2, 0)

    fetch_ids(0, 0)
    fire_gathers(0)

    def outer(cc, carry):
        for p in (0, 1):
            c = cc * 2 + p
            @pl.when(c + 1 < NCHUNK)
            def _prefetch():
                fetch_ids(c + 1, 1 - p)

                @pl.when(c >= 1)
                def _drain_out():
                    wait_out(1 - p)

                fire_gathers(1 - p)

            wait_gathers(p)
            compute_chunk(p)
            cbase = base + c * K
            pltpu.async_copy(tok_v.at[p], out_hbm.at[pl.ds(cbase, K)], sem_out[p])
        return carry

    lax.fori_loop(0, NCHUNK // 2, outer, 0)
    wait_out(0)
    wait_out(1)


def kernel(input_ids, tok_emb, pos_emb, type_emb, gamma, beta):
    comb = _comb_call(pos_emb, type_emb)
    ids = input_ids.reshape(-1).astype(jnp.int32)
    out = _sc_embed(ids, tok_emb, comb, gamma, beta)
    return out.reshape(input_ids.shape[0], input_ids.shape[1], D)
